# baseline (speedup 1.0000x reference)
import functools

import jax
import jax.numpy as jnp
from jax import lax
from jax.experimental import pallas as pl
from jax.experimental.pallas import tpu as pltpu
from jax.experimental.pallas import tpu_sc as plsc

D = 32
B = 16384
NC, NS = 2, 16
NW = NC * NS
BPW = B // NW
CHUNK = 128
NCHUNK = BPW // CHUNK
L = 16
NG = BPW // L
MARGIN = 1.0


def _sc_body(ent_hbm, rel_hbm, time_hbm, idx_hbm, scores_out, loss_out,
             idx_v, w_rows, h_rows, r_rows, t_rows, nh_rows, nt_rows,
             st, ut, wt, pos_buf, neg_buf, loss_buf, sem):
    wid = lax.axis_index("s") * NC + lax.axis_index("c")
    base = wid * BPW

    pltpu.sync_copy(idx_hbm.at[:, pl.ds(base, BPW)], idx_v)

    tables = (time_hbm, ent_hbm, rel_hbm, ent_hbm, ent_hbm, ent_hbm)
    dests = (w_rows, h_rows, r_rows, t_rows, nh_rows, nt_rows)
    copies = []
    for a in range(6):
        for j in range(NCHUNK):
            copies.append(pltpu.async_copy(
                tables[a].at[idx_v.at[a, pl.ds(j * CHUNK, CHUNK)]],
                dests[a].at[pl.ds(j * CHUNK, CHUNK)],
                sem))
    for c in copies:
        c.wait()

    zeros = jnp.zeros((L,), jnp.float32)

    def group_body(g, lacc):
        ridx = g * L + lax.iota(jnp.int32, L)
        q = zeros
        dsum = zeros
        dusum = zeros
        for d in range(D):
            cidx = jnp.full((L,), d, jnp.int32)
            w_d = plsc.load_gather(w_rows, [ridx, cidx])
            h_d = plsc.load_gather(h_rows, [ridx, cidx])
            r_d = plsc.load_gather(r_rows, [ridx, cidx])
            t_d = plsc.load_gather(t_rows, [ridx, cidx])
            x_d = plsc.load_gather(nh_rows, [ridx, cidx])
            y_d = plsc.load_gather(nt_rows, [ridx, cidx])
            s_d = h_d + r_d - t_d
            u_d = x_d + r_d - y_d
            q = q + w_d * w_d
            dsum = dsum + w_d * s_d
            dusum = dusum + w_d * u_d
            wt[pl.ds(d * L, L)] = w_d
            st[pl.ds(d * L, L)] = s_d
            ut[pl.ds(d * L, L)] = u_d
        qi = 1.0 / (q + 1e-30)
        cs = dsum * qi
        cu = dusum * qi
        pos = zeros
        neg = zeros
        for d in range(D):
            w_d = wt[pl.ds(d * L, L)]
            s_d = st[pl.ds(d * L, L)]
            u_d = ut[pl.ds(d * L, L)]
            pos = pos + jnp.abs(s_d - cs * w_d)
            neg = neg + jnp.abs(u_d - cu * w_d)
        pos_buf[pl.ds(g * L, L)] = pos
        neg_buf[pl.ds(g * L, L)] = neg
        return lacc + jnp.maximum(pos + MARGIN - neg, 0.0)

    lacc = lax.fori_loop(0, NG, group_body, zeros)

    pltpu.sync_copy(pos_buf, scores_out.at[0, pl.ds(base, BPW)])
    pltpu.sync_copy(neg_buf, scores_out.at[1, pl.ds(base, BPW)])
    loss_buf[...] = lacc
    pltpu.sync_copy(loss_buf, loss_out.at[pl.ds(wid * L, L)])


_sc_call = functools.partial(
    pl.kernel,
    out_type=(
        jax.ShapeDtypeStruct((2, B), jnp.float32),
        jax.ShapeDtypeStruct((NW * L,), jnp.float32),
    ),
    mesh=plsc.VectorSubcoreMesh(core_axis_name="c", subcore_axis_name="s"),
    compiler_params=pltpu.CompilerParams(
        needs_layout_passes=False, use_tc_tiling_on_sc=False),
    scratch_types=[
        pltpu.VMEM((6, BPW), jnp.int32),
        pltpu.VMEM((BPW, D), jnp.float32),
        pltpu.VMEM((BPW, D), jnp.float32),
        pltpu.VMEM((BPW, D), jnp.float32),
        pltpu.VMEM((BPW, D), jnp.float32),
        pltpu.VMEM((BPW, D), jnp.float32),
        pltpu.VMEM((BPW, D), jnp.float32),
        pltpu.VMEM((D * L,), jnp.float32),
        pltpu.VMEM((D * L,), jnp.float32),
        pltpu.VMEM((D * L,), jnp.float32),
        pltpu.VMEM((BPW,), jnp.float32),
        pltpu.VMEM((BPW,), jnp.float32),
        pltpu.VMEM((L,), jnp.float32),
        pltpu.SemaphoreType.DMA,
    ],
)(_sc_body)


def kernel(ent_embed, rel_embed, time_embed, pos_h, pos_r, pos_t,
           neg_h, neg_t, time_idx):
    idx_all = jnp.stack([time_idx, pos_h, pos_r, pos_t, neg_h, neg_t], axis=0)
    scores, loss_parts = _sc_call(ent_embed, rel_embed, time_embed, idx_all)
    return scores, jnp.sum(loss_parts)
